# baseline (speedup 1.0000x reference)
import jax
import jax.numpy as jnp
from jax.experimental import pallas as pl
import jax.experimental.pallas.tpu as pltpu

N = 4096
D_IN = 128
D_OUT = 128
CMAX = 512
_CHUNKS = tuple((i * 512, 512) for i in range(7)) + ((3584, 384), (3968, 128))


def _gcn_body(x_ref, w_ref, b_ref, adj_hbm, out_hbm, abuf, sup, obuf, in_sem, out_sem):
    def in_copy(i):
        st, sz = _CHUNKS[i]
        return pltpu.make_async_copy(
            adj_hbm.at[pl.ds(st, sz)],
            abuf.at[i % 2, pl.ds(0, sz)],
            in_sem.at[i % 2],
        )

    def out_copy(i):
        st, sz = _CHUNKS[i]
        return pltpu.make_async_copy(
            obuf.at[i % 2, pl.ds(0, sz)],
            out_hbm.at[pl.ds(st, sz)],
            out_sem.at[i % 2],
        )

    in_copy(0).start()
    sup[...] = jnp.dot(
        x_ref[...], w_ref[...], preferred_element_type=jnp.float32
    ).astype(jnp.bfloat16)

    n = len(_CHUNKS)
    for i in range(n):
        if i + 1 < n:
            in_copy(i + 1).start()
        in_copy(i).wait()
        if i >= 2:
            out_copy(i - 2).wait()
        sz = _CHUNKS[i][1]
        obuf[i % 2, pl.ds(0, sz)] = (
            jnp.dot(
                abuf[i % 2, pl.ds(0, sz)].astype(jnp.bfloat16),
                sup[...],
                preferred_element_type=jnp.float32,
            )
            + b_ref[...]
        )
        out_copy(i).start()
    out_copy(n - 2).wait()
    out_copy(n - 1).wait()


def kernel(x, adj, weight, bias):
    bias2d = bias.reshape(1, D_OUT)
    return pl.pallas_call(
        _gcn_body,
        in_specs=[
            pl.BlockSpec(memory_space=pltpu.MemorySpace.VMEM),
            pl.BlockSpec(memory_space=pltpu.MemorySpace.VMEM),
            pl.BlockSpec(memory_space=pltpu.MemorySpace.VMEM),
            pl.BlockSpec(memory_space=pltpu.MemorySpace.HBM),
        ],
        out_specs=pl.BlockSpec(memory_space=pltpu.MemorySpace.HBM),
        out_shape=jax.ShapeDtypeStruct((N, D_OUT), jnp.float32),
        scratch_shapes=[
            pltpu.VMEM((2, CMAX, N), jnp.float32),
            pltpu.VMEM((N, D_OUT), jnp.bfloat16),
            pltpu.VMEM((2, CMAX, D_OUT), jnp.float32),
            pltpu.SemaphoreType.DMA((2,)),
            pltpu.SemaphoreType.DMA((2,)),
        ],
    )(x, weight, bias2d, adj)
